# baseline (speedup 1.0000x reference)
import jax
import jax.numpy as jnp
from jax import lax
from jax.experimental import pallas as pl
from jax.experimental.pallas import tpu as pltpu
from jax.experimental.pallas import tpu_sc as plsc

_B, _T, _C = 16384, 200, 3
_NW = 32
_NSPLIT = 2
_ROWS_W = _B // _NSPLIT // _NW
_RB = 32
_NCHUNK = _ROWS_W // _RB
_NBUF = 3
_GROUPS = _RB * _T // 16


def _sc_body(x_hbm, tab_hbm, out_hbm, tab_v, buf, sin, sout):
    cid = lax.axis_index("c")
    sid = lax.axis_index("s")
    wid = sid * 2 + cid
    row0 = wid * _ROWS_W

    pltpu.sync_copy(tab_hbm, tab_v)

    def inv_body(k, _):
        sl = pl.ds(k * 16, 16)
        tab_v[sl] = 1.0 / tab_v[sl]
        return 0

    lax.fori_loop(0, 16, inv_body, 0)

    lanes = lax.iota(jnp.int32, 16)

    def start_in(p, c):
        pltpu.make_async_copy(
            x_hbm.at[pl.ds(row0 + c * _RB, _RB)], buf.at[p], sin.at[p]
        ).start()

    def wait_in(p, c):
        pltpu.make_async_copy(
            x_hbm.at[pl.ds(row0 + c * _RB, _RB)], buf.at[p], sin.at[p]
        ).wait()

    def start_out(p, c):
        pltpu.make_async_copy(
            buf.at[p], out_hbm.at[pl.ds(row0 + c * _RB, _RB)], sout.at[p]
        ).start()

    def wait_out(p, c):
        pltpu.make_async_copy(
            buf.at[p], out_hbm.at[pl.ds(row0 + c * _RB, _RB)], sout.at[p]
        ).wait()

    start_in(0, 0)
    start_in(1, 1)

    def chunk_body(c, _):
        p = c % _NBUF
        pvec = lanes * 0 + p
        wait_in(p, c)

        def g_body(g, rt):
            r, t = rt
            ci = t * 3 + 1
            idx = plsc.load_gather(buf, [pvec, r, ci]).astype(jnp.int32)
            nf = plsc.load_gather(tab_v, [idx])
            val = plsc.load_gather(buf, [pvec, r, ci + 1])
            plsc.store_scatter(buf, [pvec, r, ci + 1], val * nf)
            t2 = t + 16
            wrap = t2 >= _T
            return (jnp.where(wrap, r + 1, r), jnp.where(wrap, t2 - _T, t2))

        lax.fori_loop(0, _GROUPS, g_body, (lanes * 0, lanes))

        start_out(p, c)
        q = (c + 2) % _NBUF

        @pl.when(c >= 1)
        def _():
            wait_out(q, c - 1)

        @pl.when(c + 2 < _NCHUNK)
        def _():
            start_in(q, c + 2)

        return 0

    lax.fori_loop(0, _NCHUNK, chunk_body, 0)
    wait_out((_NCHUNK - 1) % _NBUF, _NCHUNK - 1)


def _sc_call_half(x, norm_factors):
    mesh = plsc.VectorSubcoreMesh(core_axis_name="c", subcore_axis_name="s")
    return pl.kernel(
        _sc_body,
        out_type=jax.ShapeDtypeStruct((_B // _NSPLIT, _T * _C), jnp.float32),
        mesh=mesh,
        scratch_types=[
            pltpu.VMEM((256,), jnp.float32),
            pltpu.VMEM((_NBUF, _RB, _T * _C), jnp.float32),
            pltpu.SemaphoreType.DMA((_NBUF,)),
            pltpu.SemaphoreType.DMA((_NBUF,)),
        ],
        compiler_params=pltpu.CompilerParams(
            needs_layout_passes=False,
        ),
    )(x, norm_factors)


@jax.jit
def kernel(td, norm_factors):
    h = _B // _NSPLIT
    ys = []
    for i in range(_NSPLIT):
        x = td[i * h:(i + 1) * h].reshape(h, _T * _C)
        ys.append(_sc_call_half(x, norm_factors).reshape(h, _T, _C))
    return jnp.concatenate(ys, axis=0)
